# Initial kernel scaffold; baseline (speedup 1.0000x reference)
import functools

import jax
import jax.numpy as jnp
from jax import lax
from jax.experimental import pallas as pl
from jax.experimental.pallas import tpu as pltpu
from jax.experimental.pallas import tpu_sc as plsc

B = 2
C = 3
N = 8192
K = 16
RATE = 2
MID = 64
OUT = 64
G = 32
IN_DIMS = 2 * C + 4

RBLK = 256
NBLK = N // RBLK
EDGES = B * N * K
EBLK = RBLK * K
TOPK = K * RATE


def _knn_kernel(a_ref, idx_ref):
    b = pl.program_id(0)
    a_full = a_ref[0]
    i = pl.program_id(1)
    a_blk = a_full[pl.ds(i * RBLK, RBLK), :]
    sq_full = jnp.sum(a_full * a_full, axis=1)
    sq_blk = jnp.sum(a_blk * a_blk, axis=1)
    dot = lax.dot_general(a_blk, a_full, (((1,), (1,)), ((), ())),
                          preferred_element_type=jnp.float32)
    d2 = (sq_blk[:, None] + sq_full[None, :]) - 2.0 * dot
    d = jnp.sqrt(jnp.maximum(d2, 0.0))

    iota = lax.broadcasted_iota(jnp.float32, (RBLK, N), 1)
    cols = []
    for t in range(TOPK - 1):
        m = jnp.min(d, axis=1, keepdims=True)
        ji = jnp.min(jnp.where(d == m, iota, float(N)), axis=1,
                     keepdims=True)
        if t % RATE == 0:
            cols.append(ji)
        if t != TOPK - 2:
            d = jnp.where(iota == ji, jnp.inf, d)
    idx_blk = jnp.concatenate(cols, axis=1)
    idx_ref[0] = idx_blk.astype(jnp.int32) + b * N


def _knn(a):
    return pl.pallas_call(
        _knn_kernel,
        grid=(B, NBLK),
        in_specs=[pl.BlockSpec((1, N, C), lambda b, i: (b, 0, 0))],
        out_specs=pl.BlockSpec((1, RBLK, K), lambda b, i: (b, i, 0)),
        out_shape=jax.ShapeDtypeStruct((B, N, K), jnp.int32),
    )(a)


_SC_CHUNK = 128
_PER_W = EDGES // 32
_NCH = _PER_W // _SC_CHUNK


def _gather_rows(table, idxf):
    info = plsc.get_sparse_core_info()
    nc, ns = info.num_cores, info.num_subcores
    idx3 = idxf.reshape(nc * ns, _NCH, _SC_CHUNK)
    mesh = plsc.VectorSubcoreMesh(core_axis_name="c", subcore_axis_name="s")

    @functools.partial(
        pl.kernel, mesh=mesh,
        out_type=jax.ShapeDtypeStruct((EDGES, 8), jnp.float32),
        scratch_types=[
            pltpu.VMEM((_NCH, _SC_CHUNK), jnp.int32),
            pltpu.VMEM((_PER_W, 8), jnp.float32),
            pltpu.SemaphoreType.DMA,
        ],
    )
    def k(table_hbm, idx_hbm, out_hbm, idx_v, rows_v, sem):
        wid = lax.axis_index("s") * nc + lax.axis_index("c")
        pltpu.sync_copy(idx_hbm.at[wid], idx_v)
        copies = []
        for j in range(_NCH):
            copies.append(pltpu.async_copy(
                table_hbm.at[idx_v.at[j]],
                rows_v.at[pl.ds(j * _SC_CHUNK, _SC_CHUNK)], sem))
        for cp in copies:
            cp.wait()
        pltpu.sync_copy(rows_v, out_hbm.at[pl.ds(wid * _PER_W, _PER_W)])

    return k(table, idx3)


def _conv1_kernel(g_ref, c_ref, w1_ref, b1_ref, h1_ref, s_ref):
    g = g_ref[...]
    c = c_ref[...]
    cr = jnp.reshape(jnp.broadcast_to(c[:, None, :], (RBLK, K, 8)),
                     (EBLK, 8))
    dx = g[:, 0:3] - cr[:, 0:3]
    dp = g[:, 3:6] - cr[:, 3:6]
    t = 1e-6 - dp
    fa = jnp.sqrt(jnp.sum(t * t, axis=1, keepdims=True))
    f10 = jnp.concatenate([dx, dp, fa, cr[:, 0:3]], axis=1)
    h1 = lax.dot_general(f10, w1_ref[...], (((1,), (1,)), ((), ())),
                         preferred_element_type=jnp.float32)
    h1 = h1 + b1_ref[...][0][None, :]
    h1_ref[...] = h1
    s1 = jnp.sum(h1, axis=0)
    s2 = jnp.sum(h1 * h1, axis=0)
    s_ref[0] = jnp.stack([s1, s2])


def _conv1(gathered, table, W1, b1):
    return pl.pallas_call(
        _conv1_kernel,
        grid=(B, NBLK),
        in_specs=[
            pl.BlockSpec((EBLK, 8), lambda b, i: (b * NBLK + i, 0)),
            pl.BlockSpec((RBLK, 8), lambda b, i: (b * NBLK + i, 0)),
            pl.BlockSpec((MID, IN_DIMS), lambda b, i: (0, 0)),
            pl.BlockSpec((1, MID), lambda b, i: (0, 0)),
        ],
        out_specs=[
            pl.BlockSpec((EBLK, MID), lambda b, i: (b * NBLK + i, 0)),
            pl.BlockSpec((1, 2, MID), lambda b, i: (b * NBLK + i, 0, 0)),
        ],
        out_shape=[
            jax.ShapeDtypeStruct((EDGES, MID), jnp.float32),
            jax.ShapeDtypeStruct((B * NBLK, 2, MID), jnp.float32),
        ],
    )(gathered, table, W1, b1.reshape(1, MID))


def _conv2_kernel(h1_ref, a_ref, w2_ref, b2_ref, h2_ref, s_ref):
    h1 = h1_ref[...]
    ab = a_ref[0]
    h = h1 * ab[0][None, :] + ab[1][None, :]
    h = jnp.where(h >= 0, h, 0.2 * h)
    h2 = lax.dot_general(h, w2_ref[...], (((1,), (1,)), ((), ())),
                         preferred_element_type=jnp.float32)
    h2 = h2 + b2_ref[...][0][None, :]
    h2_ref[...] = h2
    s1 = jnp.sum(h2, axis=0)
    s2 = jnp.sum(h2 * h2, axis=0)
    s_ref[0] = jnp.stack([s1, s2])


def _conv2(h1, ab1, W2, b2):
    return pl.pallas_call(
        _conv2_kernel,
        grid=(B, NBLK),
        in_specs=[
            pl.BlockSpec((EBLK, MID), lambda b, i: (b * NBLK + i, 0)),
            pl.BlockSpec((1, 2, MID), lambda b, i: (b, 0, 0)),
            pl.BlockSpec((OUT, MID), lambda b, i: (0, 0)),
            pl.BlockSpec((1, OUT), lambda b, i: (0, 0)),
        ],
        out_specs=[
            pl.BlockSpec((EBLK, OUT), lambda b, i: (b * NBLK + i, 0)),
            pl.BlockSpec((1, 2, OUT), lambda b, i: (b * NBLK + i, 0, 0)),
        ],
        out_shape=[
            jax.ShapeDtypeStruct((EDGES, OUT), jnp.float32),
            jax.ShapeDtypeStruct((B * NBLK, 2, OUT), jnp.float32),
        ],
    )(h1, ab1, W2, b2.reshape(1, OUT))


def _final_kernel(h2_ref, a_ref, o_ref):
    h2 = h2_ref[...]
    ab = a_ref[0]
    h = h2 * ab[0][None, :] + ab[1][None, :]
    h = jnp.where(h >= 0, h, 0.2 * h)
    h3 = jnp.reshape(h, (RBLK, K, OUT))
    o_ref[...] = jnp.max(h3, axis=1)


def _final(h2, ab2):
    return pl.pallas_call(
        _final_kernel,
        grid=(B, NBLK),
        in_specs=[
            pl.BlockSpec((EBLK, OUT), lambda b, i: (b * NBLK + i, 0)),
            pl.BlockSpec((1, 2, OUT), lambda b, i: (b, 0, 0)),
        ],
        out_specs=pl.BlockSpec((RBLK, OUT), lambda b, i: (b * NBLK + i, 0)),
        out_shape=jax.ShapeDtypeStruct((B * N, OUT), jnp.float32),
    )(h2, ab2)


def _gn_affine(stats, gamma, beta):
    cn = stats.shape[-1]
    per = cn // G
    s = stats.reshape(B, NBLK, 2, cn).sum(axis=1)
    cnt = float(per * N * K)
    sg = s.reshape(B, 2, G, per).sum(axis=3)
    mu = sg[:, 0] / cnt
    var = sg[:, 1] / cnt - mu * mu
    inv = 1.0 / jnp.sqrt(var + 1e-5)
    mu = jnp.repeat(mu, per, axis=1)
    inv = jnp.repeat(inv, per, axis=1)
    a = gamma[None, :] * inv
    bb = beta[None, :] - mu * a
    return jnp.stack([a, bb], axis=1)


def kernel(x, pos, W1, b1, g1, be1, W2, b2, g2, be2):
    a = jnp.transpose(pos, (0, 2, 1))
    xt = jnp.transpose(x, (0, 2, 1))
    table = jnp.concatenate(
        [xt, a, jnp.zeros((B, N, 2), jnp.float32)], axis=2)
    table = table.reshape(B * N, 8)

    idx = _knn(a)
    gathered = _gather_rows(table, idx.reshape(-1))

    h1, st1 = _conv1(gathered, table, W1, b1)
    ab1 = _gn_affine(st1, g1, be1)
    h2, st2 = _conv2(h1, ab1, W2, b2)
    ab2 = _gn_affine(st2, g2, be2)
    out = _final(h2, ab2)
    return jnp.transpose(out.reshape(B, N, OUT), (0, 2, 1))


# speedup vs baseline: 7.5717x; 7.5717x over previous
import functools

import jax
import jax.numpy as jnp
from jax import lax
from jax.experimental import pallas as pl
from jax.experimental.pallas import tpu as pltpu
from jax.experimental.pallas import tpu_sc as plsc

B = 2
C = 3
N = 8192
K = 16
RATE = 2
MID = 64
OUT = 64
G = 32
IN_DIMS = 2 * C + 4

RBLK = 256
NBLK = N // RBLK
EDGES = B * N * K
EBLK = RBLK * K
TOPK = K * RATE


def _knn_kernel(a_ref, idx_ref, d_s, ji_s):
    b = pl.program_id(0)
    a_full = a_ref[0]
    i = pl.program_id(1)
    a_blk = a_ref[0, pl.ds(i * RBLK, RBLK), :]
    sq_full = jnp.sum(a_full * a_full, axis=1)
    sq_blk = jnp.sum(a_blk * a_blk, axis=1)
    dot = lax.dot_general(a_blk, a_full, (((1,), (1,)), ((), ())),
                          preferred_element_type=jnp.float32)
    d2 = (sq_blk[:, None] + sq_full[None, :]) - 2.0 * dot
    d_s[...] = jnp.sqrt(jnp.maximum(d2, 0.0))

    def body(t, _):
        d = d_s[...]
        iota = lax.broadcasted_iota(jnp.int32, (RBLK, N), 1).astype(
            jnp.float32)
        m = jnp.min(d, axis=1, keepdims=True)
        ji = jnp.min(jnp.where(d == m, iota, float(N)), axis=1,
                     keepdims=True)
        ji_s[pl.ds(t, 1), :] = ji.T
        d_s[...] = jnp.where(iota == ji, jnp.inf, d)
        return 0
    lax.fori_loop(0, TOPK - 1, body, 0)

    jall = ji_s[...]
    jev = jnp.reshape(jall[: TOPK - 1 + 1], (TOPK // 2, 2, RBLK))[:, 0, :]
    idx_blk = jnp.transpose(jev, (1, 0))
    idx_ref[0] = idx_blk.astype(jnp.int32) + b * N


def _knn(a):
    return pl.pallas_call(
        _knn_kernel,
        grid=(B, NBLK),
        in_specs=[pl.BlockSpec((1, N, C), lambda b, i: (b, 0, 0))],
        out_specs=pl.BlockSpec((1, RBLK, K), lambda b, i: (b, i, 0)),
        out_shape=jax.ShapeDtypeStruct((B, N, K), jnp.int32),
        scratch_shapes=[
            pltpu.VMEM((RBLK, N), jnp.float32),
            pltpu.VMEM((TOPK, RBLK), jnp.float32),
        ],
    )(a)


_SC_CHUNK = 128
_PER_W = EDGES // 32
_NCH = _PER_W // _SC_CHUNK


def _gather_rows(table, idxf):
    info = plsc.get_sparse_core_info()
    nc, ns = info.num_cores, info.num_subcores
    idx3 = idxf.reshape(nc * ns, _NCH, _SC_CHUNK)
    mesh = plsc.VectorSubcoreMesh(core_axis_name="c", subcore_axis_name="s")

    @functools.partial(
        pl.kernel, mesh=mesh,
        compiler_params=pltpu.CompilerParams(use_tc_tiling_on_sc=False),
        out_type=jax.ShapeDtypeStruct((EDGES, 8), jnp.float32),
        scratch_types=[
            pltpu.VMEM((_NCH, _SC_CHUNK), jnp.int32),
            pltpu.VMEM((_PER_W, 8), jnp.float32),
            pltpu.SemaphoreType.DMA,
        ],
    )
    def k(table_hbm, idx_hbm, out_hbm, idx_v, rows_v, sem):
        wid = lax.axis_index("s") * nc + lax.axis_index("c")
        pltpu.sync_copy(idx_hbm.at[wid], idx_v)
        copies = []
        for j in range(_NCH):
            copies.append(pltpu.async_copy(
                table_hbm.at[idx_v.at[j]],
                rows_v.at[pl.ds(j * _SC_CHUNK, _SC_CHUNK)], sem))
        for cp in copies:
            cp.wait()
        pltpu.sync_copy(rows_v, out_hbm.at[pl.ds(wid * _PER_W, _PER_W)])

    return k(table, idx3)


def _fa_kernel(g_ref, c_ref, fa_ref):
    g = g_ref[...]
    c = c_ref[...]
    cr = jnp.reshape(jnp.broadcast_to(c[:, None, :], (RBLK, K, 8)),
                     (EBLK, 8))
    t = (cr[:, 3:6] - g[:, 3:6]) + 1e-6
    fa = jnp.sqrt(jnp.sum(t * t, axis=1))
    fa_ref[...] = jnp.reshape(fa, (RBLK, K))


def _fa_natural(gathered, table):
    return pl.pallas_call(
        _fa_kernel,
        grid=(B, NBLK),
        in_specs=[
            pl.BlockSpec((EBLK, 8), lambda b, i: (b * NBLK + i, 0)),
            pl.BlockSpec((RBLK, 8), lambda b, i: (b * NBLK + i, 0)),
        ],
        out_specs=pl.BlockSpec((RBLK, K), lambda b, i: (b * NBLK + i, 0)),
        out_shape=jax.ShapeDtypeStruct((B * N, K), jnp.float32),
    )(gathered, table)


def _conv1_kernel(g_ref, c_ref, fa_ref, w1_ref, b1_ref, h1_ref, s_ref):
    g = g_ref[...]
    c = c_ref[...]
    cr = jnp.reshape(jnp.broadcast_to(c[:, None, :], (RBLK, K, 8)),
                     (EBLK, 8))
    dx = g[:, 0:3] - cr[:, 0:3]
    dp = g[:, 3:6] - cr[:, 3:6]
    fa = fa_ref[...]
    f10 = jnp.concatenate([dx, dp, fa, cr[:, 0:3]], axis=1)
    h1 = lax.dot_general(f10, w1_ref[...], (((1,), (1,)), ((), ())),
                         preferred_element_type=jnp.float32)
    h1 = h1 + b1_ref[...][0][None, :]
    h1_ref[...] = h1
    s1 = jnp.sum(h1, axis=0)
    s2 = jnp.sum(h1 * h1, axis=0)
    s_ref[0] = jnp.stack([s1, s2])


def _conv1(gathered, table, fa_scr, W1, b1):
    return pl.pallas_call(
        _conv1_kernel,
        grid=(B, NBLK),
        in_specs=[
            pl.BlockSpec((EBLK, 8), lambda b, i: (b * NBLK + i, 0)),
            pl.BlockSpec((RBLK, 8), lambda b, i: (b * NBLK + i, 0)),
            pl.BlockSpec((EBLK, 1), lambda b, i: (b * NBLK + i, 0)),
            pl.BlockSpec((MID, IN_DIMS), lambda b, i: (0, 0)),
            pl.BlockSpec((1, MID), lambda b, i: (0, 0)),
        ],
        out_specs=[
            pl.BlockSpec((EBLK, MID), lambda b, i: (b * NBLK + i, 0)),
            pl.BlockSpec((1, 2, MID), lambda b, i: (b * NBLK + i, 0, 0)),
        ],
        out_shape=[
            jax.ShapeDtypeStruct((EDGES, MID), jnp.float32),
            jax.ShapeDtypeStruct((B * NBLK, 2, MID), jnp.float32),
        ],
    )(gathered, table, fa_scr, W1, b1.reshape(1, MID))


def _conv2_kernel(h1_ref, a_ref, w2_ref, b2_ref, h2_ref, s_ref):
    h1 = h1_ref[...]
    ab = a_ref[0]
    h = h1 * ab[0][None, :] + ab[1][None, :]
    h = jnp.where(h >= 0, h, 0.2 * h)
    h2 = lax.dot_general(h, w2_ref[...], (((1,), (1,)), ((), ())),
                         preferred_element_type=jnp.float32)
    h2 = h2 + b2_ref[...][0][None, :]
    h2_ref[...] = h2
    s1 = jnp.sum(h2, axis=0)
    s2 = jnp.sum(h2 * h2, axis=0)
    s_ref[0] = jnp.stack([s1, s2])


def _conv2(h1, ab1, W2, b2):
    return pl.pallas_call(
        _conv2_kernel,
        grid=(B, NBLK),
        in_specs=[
            pl.BlockSpec((EBLK, MID), lambda b, i: (b * NBLK + i, 0)),
            pl.BlockSpec((1, 2, MID), lambda b, i: (b, 0, 0)),
            pl.BlockSpec((OUT, MID), lambda b, i: (0, 0)),
            pl.BlockSpec((1, OUT), lambda b, i: (0, 0)),
        ],
        out_specs=[
            pl.BlockSpec((EBLK, OUT), lambda b, i: (b * NBLK + i, 0)),
            pl.BlockSpec((1, 2, OUT), lambda b, i: (b * NBLK + i, 0, 0)),
        ],
        out_shape=[
            jax.ShapeDtypeStruct((EDGES, OUT), jnp.float32),
            jax.ShapeDtypeStruct((B * NBLK, 2, OUT), jnp.float32),
        ],
    )(h1, ab1, W2, b2.reshape(1, OUT))


def _final_kernel(h2_ref, a_ref, o_ref):
    h2 = h2_ref[...]
    ab = a_ref[0]
    h = h2 * ab[0][None, :] + ab[1][None, :]
    h = jnp.where(h >= 0, h, 0.2 * h)
    h3 = jnp.reshape(h, (RBLK, K, OUT))
    o_ref[...] = jnp.max(h3, axis=1)


def _final(h2, ab2):
    return pl.pallas_call(
        _final_kernel,
        grid=(B, NBLK),
        in_specs=[
            pl.BlockSpec((EBLK, OUT), lambda b, i: (b * NBLK + i, 0)),
            pl.BlockSpec((1, 2, OUT), lambda b, i: (b, 0, 0)),
        ],
        out_specs=pl.BlockSpec((RBLK, OUT), lambda b, i: (b * NBLK + i, 0)),
        out_shape=jax.ShapeDtypeStruct((B * N, OUT), jnp.float32),
    )(h2, ab2)


def _gn_affine(stats, gamma, beta):
    cn = stats.shape[-1]
    per = cn // G
    s = stats.reshape(B, NBLK, 2, cn).sum(axis=1)
    cnt = float(per * N * K)
    sg = s.reshape(B, 2, G, per).sum(axis=3)
    mu = sg[:, 0] / cnt
    var = sg[:, 1] / cnt - mu * mu
    inv = 1.0 / jnp.sqrt(var + 1e-5)
    mu = jnp.repeat(mu, per, axis=1)
    inv = jnp.repeat(inv, per, axis=1)
    a = gamma[None, :] * inv
    bb = beta[None, :] - mu * a
    return jnp.stack([a, bb], axis=1)


def kernel(x, pos, W1, b1, g1, be1, W2, b2, g2, be2):
    a = jnp.transpose(pos, (0, 2, 1))
    xt = jnp.transpose(x, (0, 2, 1))
    table = jnp.concatenate(
        [xt, a, jnp.zeros((B, N, 2), jnp.float32)], axis=2)
    table = table.reshape(B * N, 8)

    idx = _knn(a)
    gathered = _gather_rows(table, idx.reshape(-1))

    fae = _fa_natural(gathered, table)
    fa_scr = (fae.reshape(B, N, K).transpose(0, 2, 1)
              .reshape(EDGES, 1))

    h1, st1 = _conv1(gathered, table, fa_scr, W1, b1)
    ab1 = _gn_affine(st1, g1, be1)
    h2, st2 = _conv2(h1, ab1, W2, b2)
    ab2 = _gn_affine(st2, g2, be2)
    out = _final(h2, ab2)
    return jnp.transpose(out.reshape(B, N, OUT), (0, 2, 1))
